# baseline (speedup 1.0000x reference)
import functools

import jax
import jax.numpy as jnp
from jax import lax
from jax.experimental import pallas as pl
from jax.experimental.pallas import tpu as pltpu
from jax.experimental.pallas import tpu_sc as plsc

_NC = 2
_NS = 16
_NW = _NC * _NS
_L = 16

def _build_sc_kernel(N, C, H, W):
    _WSCALE = jnp.float32(65535.0)
    _WINV = jnp.float32(1.0 / 65535.0)
    HW = H * W
    HALF = HW // 2
    CG = C * N // (_NW // 2)
    NCG = C // CG
    GCHUNK = 3136
    NGC = HALF // GCHUNK
    STAGE = 6272
    NST = HALF // STAGE
    half_w = jnp.float32((W - 1) * 0.5)
    half_h = jnp.float32((H - 1) * 0.5)

    mesh = plsc.VectorSubcoreMesh(core_axis_name="c", subcore_axis_name="s")

    @functools.partial(
        pl.kernel,
        mesh=mesh,
        compiler_params=pltpu.CompilerParams(
            use_tc_tiling_on_sc=False, needs_layout_passes=False),
        out_type=jax.ShapeDtypeStruct((N * C, HW), jnp.float32),
        scratch_types=[
            pltpu.VMEM((HALF,), jnp.int32),
            pltpu.VMEM((HALF,), jnp.int32),
            pltpu.VMEM((HW + 16 * _L,), jnp.float32),
            pltpu.VMEM((GCHUNK,), jnp.float32),
            pltpu.VMEM((GCHUNK,), jnp.float32),
            pltpu.VMEM((STAGE,), jnp.float32),
            pltpu.VMEM((STAGE,), jnp.float32),
            pltpu.SemaphoreType.DMA,
            pltpu.SemaphoreType.DMA,
        ],
    )
    def grid_sample_sc(x2, gx2, gy2, out2,
                       idx_v, wq_v, plane_v, gx_v, gy_v, st_a, st_b,
                       sem_a, sem_b):
        cid = lax.axis_index("c")
        sid = lax.axis_index("s")
        wid = sid * _NC + cid
        n = wid // (2 * NCG)
        r = wid % (2 * NCG)
        half = r // NCG
        cg = r % NCG
        row0 = n * C + cg * CG
        pxoff = half * HALF

        zeros = jnp.zeros((_L,), jnp.float32)
        for z in range(16):
            plane_v[pl.ds(HW + z * _L, _L)] = zeros

        for ch in range(NGC):
            pltpu.sync_copy(gx2.at[n, pl.ds(pxoff + ch * GCHUNK, GCHUNK)],
                            gx_v)
            pltpu.sync_copy(gy2.at[n, pl.ds(pxoff + ch * GCHUNK, GCHUNK)],
                            gy_v)

            @plsc.parallel_loop(0, GCHUNK // _L, unroll=4)
            def pre_body(gi):
                gs = pl.ds(gi * _L, _L)
                gx = gx_v[gs]
                gy = gy_v[gs]
                ix = (gx + jnp.float32(1.0)) * half_w
                iy = (gy + jnp.float32(1.0)) * half_h
                ix0 = ix.astype(jnp.int32)
                iy0 = iy.astype(jnp.int32)
                wx = ix - ix0.astype(jnp.float32)
                wy = iy - iy0.astype(jnp.float32)
                wxq = (wx * _WSCALE + jnp.float32(0.5)).astype(jnp.int32)
                wyq = (wy * _WSCALE + jnp.float32(0.5)).astype(jnp.int32)
                s = pl.ds(ch * GCHUNK + gi * _L, _L)
                idx_v[s] = iy0 * W + ix0
                wq_v[s] = wxq | (wyq << 16)

        stages = (st_a, st_b)
        sems = (sem_a, sem_b)

        def plane_body(j, carry):
            row = row0 + j
            pltpu.sync_copy(x2.at[row], plane_v.at[pl.ds(0, HW)])
            for st in range(NST):
                stv = stages[st % 2]
                sem = sems[st % 2]
                if st < 2:
                    @pl.when(j > 0)
                    def _drain():
                        pltpu.make_async_copy(
                            stv, out2.at[row0, pl.ds(pxoff + st * STAGE,
                                                     STAGE)], sem).wait()
                else:
                    descs[st % 2].wait()

                @plsc.parallel_loop(0, STAGE // _L, unroll=4)
                def lerp_body(gi):
                    s = pl.ds(st * STAGE + gi * _L, _L)
                    i00 = idx_v[s]
                    wq = wq_v[s]
                    i10 = i00 + 1
                    i01 = i00 + W
                    i11 = i00 + (W + 1)
                    wx = jnp.bitwise_and(wq, 0xFFFF).astype(jnp.float32) * _WINV
                    wy = lax.shift_right_logical(wq, 16).astype(jnp.float32) * _WINV
                    v00 = plsc.load_gather(plane_v, [i00])
                    v10 = plsc.load_gather(plane_v, [i10])
                    v01 = plsc.load_gather(plane_v, [i01])
                    v11 = plsc.load_gather(plane_v, [i11])
                    top = v00 + wx * (v10 - v00)
                    bot = v01 + wx * (v11 - v01)
                    stv[pl.ds(gi * _L, _L)] = top + wy * (bot - top)
                d = pltpu.async_copy(
                    stv, out2.at[row, pl.ds(pxoff + st * STAGE, STAGE)], sem)
                if st < 2:
                    descs[st % 2] = d
            return carry

        descs = [None, None]
        lax.fori_loop(0, CG, plane_body, 0)
        for b in range(2):
            pltpu.make_async_copy(
                stages[b], out2.at[row0, pl.ds(pxoff, STAGE)], sems[b]).wait()

    return grid_sample_sc


def kernel(x, g):
    N, C, H, W = x.shape
    Ho, Wo = g.shape[1], g.shape[2]
    gx2 = g[..., 0].reshape(N, Ho * Wo)
    gy2 = g[..., 1].reshape(N, Ho * Wo)
    NB = N // 4
    sc = _build_sc_kernel(NB, C, H, W)
    halves = []
    for i in range(4):
        xi = x[i * NB:(i + 1) * NB].reshape(NB * C, H * W)
        oi = sc(xi, gx2[i * NB:(i + 1) * NB], gy2[i * NB:(i + 1) * NB])
        halves.append(oi.reshape(NB, C, H, W))
    return jnp.concatenate(halves, axis=0)


# speedup vs baseline: 1.0543x; 1.0543x over previous
import functools

import jax
import jax.numpy as jnp
from jax import lax
from jax.experimental import pallas as pl
from jax.experimental.pallas import tpu as pltpu
from jax.experimental.pallas import tpu_sc as plsc

_NC = 2
_NS = 16
_NW = _NC * _NS
_L = 16

def _build_sc_kernel(N, C, H, W):
    _WSCALE = jnp.float32(65535.0)
    _WINV = jnp.float32(1.0 / 65535.0)
    HW = H * W
    HALF = HW // 2
    CG = C * N // (_NW // 2)
    NCG = C // CG
    GCHUNK = 3136
    NGC = HALF // GCHUNK
    STAGE = 6272
    NST = HALF // STAGE
    half_w = jnp.float32((W - 1) * 0.5)
    half_h = jnp.float32((H - 1) * 0.5)

    mesh = plsc.VectorSubcoreMesh(core_axis_name="c", subcore_axis_name="s")

    @functools.partial(
        pl.kernel,
        mesh=mesh,
        compiler_params=pltpu.CompilerParams(
            use_tc_tiling_on_sc=False, needs_layout_passes=False),
        out_type=jax.ShapeDtypeStruct((N * C, HW), jnp.float32),
        scratch_types=[
            pltpu.VMEM((HALF,), jnp.int32),
            pltpu.VMEM((HALF,), jnp.int32),
            pltpu.VMEM((HW + 16 * _L,), jnp.float32),
            pltpu.VMEM((GCHUNK,), jnp.float32),
            pltpu.VMEM((GCHUNK,), jnp.float32),
            pltpu.VMEM((STAGE,), jnp.float32),
            pltpu.VMEM((STAGE,), jnp.float32),
            pltpu.SemaphoreType.DMA,
            pltpu.SemaphoreType.DMA,
        ],
    )
    def grid_sample_sc(x2, gx2, gy2, out2,
                       idx_v, wq_v, plane_v, gx_v, gy_v, st_a, st_b,
                       sem_a, sem_b):
        cid = lax.axis_index("c")
        sid = lax.axis_index("s")
        wid = sid * _NC + cid
        n = wid // (2 * NCG)
        r = wid % (2 * NCG)
        half = r // NCG
        cg = r % NCG
        row0 = n * C + cg * CG
        pxoff = half * HALF

        zeros = jnp.zeros((_L,), jnp.float32)
        for z in range(16):
            plane_v[pl.ds(HW + z * _L, _L)] = zeros

        for ch in range(NGC):
            pltpu.sync_copy(gx2.at[n, pl.ds(pxoff + ch * GCHUNK, GCHUNK)],
                            gx_v)
            pltpu.sync_copy(gy2.at[n, pl.ds(pxoff + ch * GCHUNK, GCHUNK)],
                            gy_v)

            @plsc.parallel_loop(0, GCHUNK // _L, unroll=4)
            def pre_body(gi):
                gs = pl.ds(gi * _L, _L)
                gx = gx_v[gs]
                gy = gy_v[gs]
                ix = (gx + jnp.float32(1.0)) * half_w
                iy = (gy + jnp.float32(1.0)) * half_h
                ix0 = ix.astype(jnp.int32)
                iy0 = iy.astype(jnp.int32)
                wx = ix - ix0.astype(jnp.float32)
                wy = iy - iy0.astype(jnp.float32)
                wxq = (wx * _WSCALE + jnp.float32(0.5)).astype(jnp.int32)
                wyq = (wy * _WSCALE + jnp.float32(0.5)).astype(jnp.int32)
                s = pl.ds(ch * GCHUNK + gi * _L, _L)
                idx_v[s] = iy0 * W + ix0
                wq_v[s] = wxq | (wyq << 16)

        stages = (st_a, st_b)
        sems = (sem_a, sem_b)

        def plane_body(j, carry):
            row = row0 + j
            pltpu.sync_copy(x2.at[row], plane_v.at[pl.ds(0, HW)])
            for st in range(NST):
                stv = stages[st % 2]
                sem = sems[st % 2]
                if st < 2:
                    @pl.when(j > 0)
                    def _drain():
                        pltpu.make_async_copy(
                            stv, out2.at[row0, pl.ds(pxoff + st * STAGE,
                                                     STAGE)], sem).wait()
                else:
                    descs[st % 2].wait()

                @plsc.parallel_loop(0, STAGE // _L, unroll=6)
                def lerp_body(gi):
                    s = pl.ds(st * STAGE + gi * _L, _L)
                    i00 = idx_v[s]
                    wq = wq_v[s]
                    i10 = i00 + 1
                    i01 = i00 + W
                    i11 = i00 + (W + 1)
                    wx = jnp.bitwise_and(wq, 0xFFFF).astype(jnp.float32) * _WINV
                    wy = lax.shift_right_logical(wq, 16).astype(jnp.float32) * _WINV
                    v00 = plsc.load_gather(plane_v, [i00])
                    v10 = plsc.load_gather(plane_v, [i10])
                    v01 = plsc.load_gather(plane_v, [i01])
                    v11 = plsc.load_gather(plane_v, [i11])
                    top = v00 + wx * (v10 - v00)
                    bot = v01 + wx * (v11 - v01)
                    stv[pl.ds(gi * _L, _L)] = top + wy * (bot - top)
                d = pltpu.async_copy(
                    stv, out2.at[row, pl.ds(pxoff + st * STAGE, STAGE)], sem)
                if st < 2:
                    descs[st % 2] = d
            return carry

        descs = [None, None]
        lax.fori_loop(0, CG, plane_body, 0)
        for b in range(2):
            pltpu.make_async_copy(
                stages[b], out2.at[row0, pl.ds(pxoff, STAGE)], sems[b]).wait()

    return grid_sample_sc


def kernel(x, g):
    N, C, H, W = x.shape
    Ho, Wo = g.shape[1], g.shape[2]
    gx2 = g[..., 0].reshape(N, Ho * Wo)
    gy2 = g[..., 1].reshape(N, Ho * Wo)
    NB = N // 2
    sc = _build_sc_kernel(NB, C, H, W)
    halves = []
    for i in range(2):
        xi = x[i * NB:(i + 1) * NB].reshape(NB * C, H * W)
        oi = sc(xi, gx2[i * NB:(i + 1) * NB], gy2[i * NB:(i + 1) * NB])
        halves.append(oi.reshape(NB, C, H, W))
    return jnp.concatenate(halves, axis=0)


# speedup vs baseline: 1.0574x; 1.0029x over previous
import functools

import jax
import jax.numpy as jnp
from jax import lax
from jax.experimental import pallas as pl
from jax.experimental.pallas import tpu as pltpu
from jax.experimental.pallas import tpu_sc as plsc

_NC = 2
_NS = 16
_NW = _NC * _NS
_L = 16

def _build_sc_kernel(N, C, H, W):
    _WSCALE = jnp.float32(65535.0)
    _WINV = jnp.float32(1.0 / 65535.0)
    HW = H * W
    HALF = HW // 2
    CG = C * N // (_NW // 2)
    NCG = C // CG
    GCHUNK = 3136
    NGC = HALF // GCHUNK
    STAGE = 6272
    NST = HALF // STAGE
    half_w = jnp.float32((W - 1) * 0.5)
    half_h = jnp.float32((H - 1) * 0.5)

    mesh = plsc.VectorSubcoreMesh(core_axis_name="c", subcore_axis_name="s")

    @functools.partial(
        pl.kernel,
        mesh=mesh,
        compiler_params=pltpu.CompilerParams(
            use_tc_tiling_on_sc=False, needs_layout_passes=False),
        out_type=jax.ShapeDtypeStruct((N * C, HW), jnp.float32),
        scratch_types=[
            pltpu.VMEM((HALF,), jnp.int32),
            pltpu.VMEM((HALF,), jnp.int32),
            pltpu.VMEM((HW + 16 * _L,), jnp.float32),
            pltpu.VMEM((GCHUNK,), jnp.float32),
            pltpu.VMEM((GCHUNK,), jnp.float32),
            pltpu.VMEM((STAGE,), jnp.float32),
            pltpu.VMEM((STAGE,), jnp.float32),
            pltpu.SemaphoreType.DMA,
            pltpu.SemaphoreType.DMA,
        ],
    )
    def grid_sample_sc(x2, gx2, gy2, out2,
                       idx_v, wq_v, plane_v, gx_v, gy_v, st_a, st_b,
                       sem_a, sem_b):
        cid = lax.axis_index("c")
        sid = lax.axis_index("s")
        wid = sid * _NC + cid
        n = wid // (2 * NCG)
        r = wid % (2 * NCG)
        half = r // NCG
        cg = r % NCG
        row0 = n * C + cg * CG
        pxoff = half * HALF

        zeros = jnp.zeros((_L,), jnp.float32)
        for z in range(16):
            plane_v[pl.ds(HW + z * _L, _L)] = zeros

        for ch in range(NGC):
            pltpu.sync_copy(gx2.at[n, pl.ds(pxoff + ch * GCHUNK, GCHUNK)],
                            gx_v)
            pltpu.sync_copy(gy2.at[n, pl.ds(pxoff + ch * GCHUNK, GCHUNK)],
                            gy_v)

            @plsc.parallel_loop(0, GCHUNK // _L, unroll=4)
            def pre_body(gi):
                gs = pl.ds(gi * _L, _L)
                gx = gx_v[gs]
                gy = gy_v[gs]
                ix = (gx + jnp.float32(1.0)) * half_w
                iy = (gy + jnp.float32(1.0)) * half_h
                ix0 = ix.astype(jnp.int32)
                iy0 = iy.astype(jnp.int32)
                wx = ix - ix0.astype(jnp.float32)
                wy = iy - iy0.astype(jnp.float32)
                wxq = (wx * _WSCALE + jnp.float32(0.5)).astype(jnp.int32)
                wyq = (wy * _WSCALE + jnp.float32(0.5)).astype(jnp.int32)
                s = pl.ds(ch * GCHUNK + gi * _L, _L)
                idx_v[s] = iy0 * W + ix0
                wq_v[s] = wxq | (wyq << 16)

        stages = (st_a, st_b)
        sems = (sem_a, sem_b)

        def plane_body(j, carry):
            row = row0 + j
            pltpu.sync_copy(x2.at[row], plane_v.at[pl.ds(0, HW)])
            for st in range(NST):
                stv = stages[st % 2]
                sem = sems[st % 2]
                if st < 2:
                    @pl.when(j > 0)
                    def _drain():
                        pltpu.make_async_copy(
                            stv, out2.at[row0, pl.ds(pxoff + st * STAGE,
                                                     STAGE)], sem).wait()
                else:
                    descs[st % 2].wait()

                @plsc.parallel_loop(0, STAGE // _L, unroll=4)
                def lerp_body(gi):
                    s = pl.ds(st * STAGE + gi * _L, _L)
                    i00 = idx_v[s]
                    wq = wq_v[s]
                    i10 = i00 + 1
                    i01 = i00 + W
                    i11 = i00 + (W + 1)
                    wx = jnp.bitwise_and(wq, 0xFFFF).astype(jnp.float32) * _WINV
                    wy = lax.shift_right_logical(wq, 16).astype(jnp.float32) * _WINV
                    v00 = plsc.load_gather(plane_v, [i00])
                    v10 = plsc.load_gather(plane_v, [i10])
                    v01 = plsc.load_gather(plane_v, [i01])
                    v11 = plsc.load_gather(plane_v, [i11])
                    top = v00 + wx * (v10 - v00)
                    bot = v01 + wx * (v11 - v01)
                    stv[pl.ds(gi * _L, _L)] = top + wy * (bot - top)
                d = pltpu.async_copy(
                    stv, out2.at[row, pl.ds(pxoff + st * STAGE, STAGE)], sem)
                if st < 2:
                    descs[st % 2] = d
            return carry

        descs = [None, None]
        lax.fori_loop(0, CG, plane_body, 0)
        for b in range(2):
            pltpu.make_async_copy(
                stages[b], out2.at[row0, pl.ds(pxoff, STAGE)], sems[b]).wait()

    return grid_sample_sc


def kernel(x, g):
    N, C, H, W = x.shape
    Ho, Wo = g.shape[1], g.shape[2]
    gx2 = g[..., 0].reshape(N, Ho * Wo)
    gy2 = g[..., 1].reshape(N, Ho * Wo)
    NB = N // 2
    sc = _build_sc_kernel(NB, C, H, W)
    halves = []
    for i in range(2):
        xi = x[i * NB:(i + 1) * NB].reshape(NB * C, H * W)
        oi = sc(xi, gx2[i * NB:(i + 1) * NB], gy2[i * NB:(i + 1) * NB])
        halves.append(oi.reshape(NB, C, H, W))
    return jnp.concatenate(halves, axis=0)
